# baseline (speedup 1.0000x reference)
import functools

import jax
import jax.numpy as jnp
from jax import lax
from jax.experimental import pallas as pl
from jax.experimental.pallas import tpu as pltpu
from jax.experimental.pallas import tpu_sc as plsc

NC = 2
NS = 16
NW = NC * NS

BATCH = 4096
SEQ = 200
DIM = 64

NPT = BATCH // NW
CHUNKS = (104, 96)
OFFS = (0, 104)
LANES = 16
NACC = DIM // (2 * LANES)

_mesh = plsc.VectorSubcoreMesh(core_axis_name="c", subcore_axis_name="s")


@functools.partial(
    pl.kernel,
    out_type=jax.ShapeDtypeStruct((BATCH * SEQ,), jnp.int32),
    mesh=_mesh,
    compiler_params=pltpu.CompilerParams(use_tc_tiling_on_sc=True,
                                         needs_layout_passes=False),
    scratch_types=[
        pltpu.VMEM((SEQ, NPT), jnp.int32),
        pltpu.VMEM((NPT * SEQ,), jnp.int32),
    ],
)
def _detile_idx(tgt_hbm, out_hbm, stg_v, idx_v):
    wid = lax.axis_index("s") * NC + lax.axis_index("c")
    pltpu.sync_copy(tgt_hbm.at[:, pl.ds(wid * NPT, NPT)], stg_v)
    lane_seq = SEQ * lax.iota(jnp.int32, LANES)

    def tbody(s, _):
        for m in range(NPT // LANES):
            plsc.store_scatter(idx_v, [m * LANES * SEQ + lane_seq + s],
                               stg_v[s, pl.ds(m * LANES, LANES)])
        return _

    lax.fori_loop(0, SEQ, tbody, None, unroll=4)
    pltpu.sync_copy(idx_v, out_hbm.at[pl.ds(wid * NPT * SEQ, NPT * SEQ)])


@functools.partial(
    pl.kernel,
    out_type=jax.ShapeDtypeStruct((BATCH * DIM,), jnp.float32),
    mesh=_mesh,
    compiler_params=pltpu.CompilerParams(use_tc_tiling_on_sc=False,
                                         needs_layout_passes=False),
    scratch_types=[
        pltpu.VMEM((NPT * SEQ,), jnp.int32),
        pltpu.VMEM((SEQ, DIM), jnp.bfloat16),
        pltpu.VMEM((SEQ, DIM), jnp.bfloat16),
        pltpu.VMEM((NPT + LANES,), jnp.int32),
        pltpu.VMEM((NPT * DIM,), jnp.float32),
        pltpu.SemaphoreType.DMA,
        pltpu.SemaphoreType.DMA,
    ],
)
def _bow_pool(table_hbm, tgt_hbm, len_hbm, out_hbm,
              idx_v, rows0, rows1, len_v, out_v, sem0, sem1):
    wid = lax.axis_index("s") * NC + lax.axis_index("c")
    base = wid * NPT

    pltpu.sync_copy(tgt_hbm.at[pl.ds(base * SEQ, NPT * SEQ)], idx_v)
    pltpu.sync_copy(len_hbm.at[pl.ds(base, NPT)], len_v.at[pl.ds(0, NPT)])

    def gather(elem, rows_ref, sem, j):
        return pltpu.make_async_copy(
            table_hbm.at[idx_v.at[pl.ds(elem * SEQ + OFFS[j], CHUNKS[j])]],
            rows_ref.at[pl.ds(OFFS[j], CHUNKS[j])], sem)

    def issue(elem, rows_ref, sem):
        for j in range(2):
            gather(elem, rows_ref, sem, j).start()

    def wait(elem, rows_ref, sem):
        for j in range(2):
            gather(elem, rows_ref, sem, j).wait()

    def compute(elem, rows_ref):
        BLK = 8

        def blk(bi, accs):
            r0 = bi * BLK
            p = [rows_ref[r0, pl.ds(c * 2 * LANES, 2 * LANES)]
                 for c in range(NACC)]
            for k in range(1, BLK):
                for c in range(NACC):
                    p[c] = p[c] + rows_ref[r0 + k,
                                           pl.ds(c * 2 * LANES, 2 * LANES)]
            out = []
            for c in range(NACC):
                ev, od = plsc.unpack(p[c],
                                     format=plsc.PackFormat.INTERLEAVED)
                out.extend((accs[2 * c] + ev, accs[2 * c + 1] + od))
            return tuple(out)

        zeros = tuple(jnp.zeros((LANES,), jnp.float32) for _ in range(2 * NACC))
        accs = lax.fori_loop(0, SEQ // BLK, blk, zeros, unroll=2)
        lvv = len_v[pl.ds(elem, LANES)].astype(jnp.float32)
        inv = jnp.full((LANES,), 1.0, jnp.float32) / lvv
        scale = inv[0]
        row_base = elem * DIM
        lane2 = 2 * lax.iota(jnp.int32, LANES)
        for c in range(NACC):
            plsc.store_scatter(out_v, [row_base + lane2 + 2 * c * LANES],
                               accs[2 * c] * scale)
            plsc.store_scatter(out_v, [row_base + lane2 + 2 * c * LANES + 1],
                               accs[2 * c + 1] * scale)

    issue(0, rows0, sem0)
    issue(1, rows1, sem1)

    def body(k, _):
        i = 2 * k
        wait(i, rows0, sem0)
        compute(i, rows0)
        issue(i + 2, rows0, sem0)
        wait(i + 1, rows1, sem1)
        compute(i + 1, rows1)
        issue(i + 3, rows1, sem1)
        return _

    lax.fori_loop(0, NPT // 2 - 1, body, None)

    wait(NPT - 2, rows0, sem0)
    compute(NPT - 2, rows0)
    wait(NPT - 1, rows1, sem1)
    compute(NPT - 1, rows1)

    pltpu.sync_copy(out_v, out_hbm.at[pl.ds(base * DIM, NPT * DIM)])


def kernel(embed_table, target, target_length):
    idx_flat = _detile_idx(target.astype(jnp.int32).T)
    out = _bow_pool(embed_table.astype(jnp.bfloat16),
                    idx_flat,
                    target_length.astype(jnp.int32))
    return out.reshape(BATCH, DIM)


# speedup vs baseline: 1.0535x; 1.0535x over previous
import functools

import jax
import jax.numpy as jnp
from jax import lax
from jax.experimental import pallas as pl
from jax.experimental.pallas import tpu as pltpu
from jax.experimental.pallas import tpu_sc as plsc

NC = 2
NS = 16
NW = NC * NS

BATCH = 4096
SEQ = 200
DIM = 64

NPT = BATCH // NW
CHUNKS = (104, 96)
OFFS = (0, 104)
LANES = 16
NACC = DIM // (2 * LANES)

_mesh = plsc.VectorSubcoreMesh(core_axis_name="c", subcore_axis_name="s")


@functools.partial(
    pl.kernel,
    out_type=jax.ShapeDtypeStruct((BATCH * DIM,), jnp.float32),
    mesh=_mesh,
    compiler_params=pltpu.CompilerParams(use_tc_tiling_on_sc=False,
                                         needs_layout_passes=False),
    scratch_types=[
        pltpu.VMEM((NPT * SEQ,), jnp.int32),
        pltpu.VMEM((SEQ, DIM), jnp.bfloat16),
        pltpu.VMEM((SEQ, DIM), jnp.bfloat16),
        pltpu.VMEM((NPT + LANES,), jnp.int32),
        pltpu.VMEM((NPT * DIM,), jnp.float32),
        pltpu.SemaphoreType.DMA,
        pltpu.SemaphoreType.DMA,
    ],
)
def _bow_pool(table_hbm, tgt_hbm, len_hbm, out_hbm,
              idx_v, rows0, rows1, len_v, out_v, sem0, sem1):
    wid = lax.axis_index("s") * NC + lax.axis_index("c")
    base = wid * NPT

    pltpu.sync_copy(tgt_hbm.at[pl.ds(base * SEQ, NPT * SEQ)], idx_v)
    pltpu.sync_copy(len_hbm.at[pl.ds(base, NPT)], len_v.at[pl.ds(0, NPT)])

    def gather(elem, rows_ref, sem, j):
        return pltpu.make_async_copy(
            table_hbm.at[idx_v.at[pl.ds(elem * SEQ + OFFS[j], CHUNKS[j])]],
            rows_ref.at[pl.ds(OFFS[j], CHUNKS[j])], sem)

    def issue(elem, rows_ref, sem):
        for j in range(2):
            gather(elem, rows_ref, sem, j).start()

    def wait(elem, rows_ref, sem):
        for j in range(2):
            gather(elem, rows_ref, sem, j).wait()

    def compute(elem, rows_ref):
        BLK = 8

        def blk(bi, accs):
            r0 = bi * BLK
            p = [rows_ref[r0, pl.ds(c * 2 * LANES, 2 * LANES)]
                 for c in range(NACC)]
            for k in range(1, BLK):
                for c in range(NACC):
                    p[c] = p[c] + rows_ref[r0 + k,
                                           pl.ds(c * 2 * LANES, 2 * LANES)]
            out = []
            for c in range(NACC):
                ev, od = plsc.unpack(p[c],
                                     format=plsc.PackFormat.INTERLEAVED)
                out.extend((accs[2 * c] + ev, accs[2 * c + 1] + od))
            return tuple(out)

        zeros = tuple(jnp.zeros((LANES,), jnp.float32) for _ in range(2 * NACC))
        accs = lax.fori_loop(0, SEQ // BLK, blk, zeros, unroll=5)
        lvv = len_v[pl.ds(elem, LANES)].astype(jnp.float32)
        inv = jnp.full((LANES,), 1.0, jnp.float32) / lvv
        scale = inv[0]
        row_base = elem * DIM
        lane2 = 2 * lax.iota(jnp.int32, LANES)
        for c in range(NACC):
            plsc.store_scatter(out_v, [row_base + lane2 + 2 * c * LANES],
                               accs[2 * c] * scale)
            plsc.store_scatter(out_v, [row_base + lane2 + 2 * c * LANES + 1],
                               accs[2 * c + 1] * scale)

    issue(0, rows0, sem0)
    issue(1, rows1, sem1)

    def body(k, _):
        i = 2 * k
        wait(i, rows0, sem0)
        compute(i, rows0)
        issue(i + 2, rows0, sem0)
        wait(i + 1, rows1, sem1)
        compute(i + 1, rows1)
        issue(i + 3, rows1, sem1)
        return _

    lax.fori_loop(0, NPT // 2 - 1, body, None)

    wait(NPT - 2, rows0, sem0)
    compute(NPT - 2, rows0)
    wait(NPT - 1, rows1, sem1)
    compute(NPT - 1, rows1)

    pltpu.sync_copy(out_v, out_hbm.at[pl.ds(base * DIM, NPT * DIM)])


def kernel(embed_table, target, target_length):
    out = _bow_pool(embed_table.astype(jnp.bfloat16),
                    target.astype(jnp.int32).reshape(BATCH * SEQ),
                    target_length.astype(jnp.int32))
    return out.reshape(BATCH, DIM)


# speedup vs baseline: 1.0607x; 1.0068x over previous
import functools

import jax
import jax.numpy as jnp
from jax import lax
from jax.experimental import pallas as pl
from jax.experimental.pallas import tpu as pltpu
from jax.experimental.pallas import tpu_sc as plsc

NC = 2
NS = 16
NW = NC * NS

BATCH = 4096
SEQ = 200
DIM = 64

NPT = BATCH // NW
CHUNKS = (104, 96)
OFFS = (0, 104)
LANES = 16
NACC = DIM // (2 * LANES)

_mesh = plsc.VectorSubcoreMesh(core_axis_name="c", subcore_axis_name="s")


@functools.partial(
    pl.kernel,
    out_type=jax.ShapeDtypeStruct((BATCH * DIM,), jnp.float32),
    mesh=_mesh,
    compiler_params=pltpu.CompilerParams(use_tc_tiling_on_sc=False,
                                         needs_layout_passes=False),
    scratch_types=[
        pltpu.VMEM((NPT * SEQ,), jnp.int32),
        pltpu.VMEM((SEQ, DIM), jnp.bfloat16),
        pltpu.VMEM((SEQ, DIM), jnp.bfloat16),
        pltpu.VMEM((NPT + LANES,), jnp.int32),
        pltpu.VMEM((NPT * DIM,), jnp.float32),
        pltpu.SemaphoreType.DMA,
        pltpu.SemaphoreType.DMA,
    ],
)
def _bow_pool(table_hbm, tgt_hbm, len_hbm, out_hbm,
              idx_v, rows0, rows1, len_v, out_v, sem0, sem1):
    wid = lax.axis_index("s") * NC + lax.axis_index("c")
    base = wid * NPT

    pltpu.sync_copy(tgt_hbm.at[pl.ds(base * SEQ, NPT * SEQ)], idx_v)
    pltpu.sync_copy(len_hbm.at[pl.ds(base, NPT)], len_v.at[pl.ds(0, NPT)])

    def gather(elem, rows_ref, sem, j):
        return pltpu.make_async_copy(
            table_hbm.at[idx_v.at[pl.ds(elem * SEQ + OFFS[j], CHUNKS[j])]],
            rows_ref.at[pl.ds(OFFS[j], CHUNKS[j])], sem)

    def issue(elem, rows_ref, sem):
        for j in range(2):
            gather(elem, rows_ref, sem, j).start()

    def wait(elem, rows_ref, sem):
        for j in range(2):
            gather(elem, rows_ref, sem, j).wait()

    def compute(elem, rows_ref):
        BLK = 8

        def blk(bi, accs):
            r0 = bi * BLK
            p = [rows_ref[r0, pl.ds(c * 2 * LANES, 2 * LANES)]
                 for c in range(NACC)]
            for k in range(1, BLK):
                for c in range(NACC):
                    p[c] = p[c] + rows_ref[r0 + k,
                                           pl.ds(c * 2 * LANES, 2 * LANES)]
            out = []
            for c in range(NACC):
                ev, od = plsc.unpack(p[c],
                                     format=plsc.PackFormat.INTERLEAVED)
                out.extend((accs[2 * c] + ev, accs[2 * c + 1] + od))
            return tuple(out)

        zeros = tuple(jnp.zeros((LANES,), jnp.float32) for _ in range(2 * NACC))
        accs = lax.fori_loop(0, SEQ // BLK, blk, zeros, unroll=2)
        lvv = len_v[pl.ds(elem, LANES)].astype(jnp.float32)
        inv = jnp.full((LANES,), 1.0, jnp.float32) / lvv
        scale = inv[0]
        row_base = elem * DIM
        lane2 = 2 * lax.iota(jnp.int32, LANES)
        for c in range(NACC):
            plsc.store_scatter(out_v, [row_base + lane2 + 2 * c * LANES],
                               accs[2 * c] * scale)
            plsc.store_scatter(out_v, [row_base + lane2 + 2 * c * LANES + 1],
                               accs[2 * c + 1] * scale)

    issue(0, rows0, sem0)
    issue(1, rows1, sem1)

    def body(k, _):
        i = 2 * k
        wait(i, rows0, sem0)
        compute(i, rows0)
        issue(i + 2, rows0, sem0)
        wait(i + 1, rows1, sem1)
        compute(i + 1, rows1)
        issue(i + 3, rows1, sem1)
        return _

    lax.fori_loop(0, NPT // 2 - 1, body, None)

    wait(NPT - 2, rows0, sem0)
    compute(NPT - 2, rows0)
    wait(NPT - 1, rows1, sem1)
    compute(NPT - 1, rows1)

    pltpu.sync_copy(out_v, out_hbm.at[pl.ds(base * DIM, NPT * DIM)])


def kernel(embed_table, target, target_length):
    out = _bow_pool(embed_table.astype(jnp.bfloat16),
                    target.astype(jnp.int32).reshape(BATCH * SEQ),
                    target_length.astype(jnp.int32))
    return out.reshape(BATCH, DIM)
